# Initial kernel scaffold; baseline (speedup 1.0000x reference)
import jax
import jax.numpy as jnp
from jax.experimental import pallas as pl


def kernel(input, weight):
    raise NotImplementedError("write your pallas kernel here")


# speedup vs baseline: 1.4978x; 1.4978x over previous
import functools

import jax
import jax.numpy as jnp
from jax import lax
from jax.experimental import pallas as pl
from jax.experimental.pallas import tpu as pltpu
from jax.experimental.pallas import tpu_sc as plsc

EMB_D = 32


@functools.lru_cache(maxsize=None)
def _sc_geometry():
    try:
        info = plsc.get_sparse_core_info()
        return int(info.num_cores), int(info.num_subcores)
    except Exception:
        return 2, 16


@functools.lru_cache(maxsize=None)
def _make_gather(vocab: int, batch: int, chunk: int):
    nc, ns = _sc_geometry()
    nw = nc * ns
    b_per_w = batch // nw
    n_pairs = b_per_w // (2 * chunk)
    assert b_per_w % (2 * chunk) == 0 and chunk % 8 == 0

    mesh = plsc.VectorSubcoreMesh(core_axis_name="c", subcore_axis_name="s")

    @functools.partial(
        pl.kernel,
        mesh=mesh,
        out_type=jax.ShapeDtypeStruct((batch, EMB_D), jnp.float32),
        scratch_types=[
            pltpu.VMEM((chunk,), jnp.int32),
            pltpu.VMEM((chunk,), jnp.int32),
            pltpu.VMEM((chunk, EMB_D), jnp.float32),
            pltpu.VMEM((chunk, EMB_D), jnp.float32),
            pltpu.SemaphoreType.DMA,
            pltpu.SemaphoreType.DMA,
            pltpu.SemaphoreType.DMA,
            pltpu.SemaphoreType.DMA,
        ],
        compiler_params=pltpu.CompilerParams(use_tc_tiling_on_sc=False),
    )
    def gather_kernel(table_hbm, idx_hbm, out_hbm, idx_a, idx_b, rows_a, rows_b,
                      sem_ga, sem_gb, sem_wa, sem_wb):
        wid = lax.axis_index("s") * nc + lax.axis_index("c")
        base = wid * b_per_w

        def pair(j, carry):
            off_a = base + (2 * j) * chunk
            off_b = off_a + chunk
            pltpu.sync_copy(idx_hbm.at[pl.ds(off_a, chunk)], idx_a)
            ga = pltpu.async_copy(table_hbm.at[idx_a], rows_a, sem_ga)
            pltpu.sync_copy(idx_hbm.at[pl.ds(off_b, chunk)], idx_b)
            gb = pltpu.async_copy(table_hbm.at[idx_b], rows_b, sem_gb)
            ga.wait()
            wa = pltpu.async_copy(rows_a, out_hbm.at[pl.ds(off_a, chunk)], sem_wa)
            gb.wait()
            wb = pltpu.async_copy(rows_b, out_hbm.at[pl.ds(off_b, chunk)], sem_wb)
            wa.wait()
            wb.wait()
            return carry

        lax.fori_loop(0, n_pairs, pair, 0)

    return gather_kernel


def kernel(input, weight):
    b, s = input.shape
    batch = b * s
    idx = input.reshape(batch).astype(jnp.int32)
    out = _make_gather(weight.shape[0], batch, 1600)(weight, idx)
    return out.reshape(b, s, EMB_D)
